# Initial kernel scaffold; baseline (speedup 1.0000x reference)
import jax
import jax.numpy as jnp
from jax.experimental import pallas as pl


def kernel(A, X, E, We, be, Wn, bn, Wc, bc, eps, Wo, bo):
    raise NotImplementedError("write your pallas kernel here")


# speedup vs baseline: 2.7517x; 2.7517x over previous
import jax
import jax.numpy as jnp
from jax.experimental import pallas as pl

B, N, D_IN, D_EDGE, D, L = 2, 512, 5, 1, 64, 2
D_IN_PAD = 8


def _gnn_kernel(a_ref, e_ref, x_ref, wn_ref, bn_ref, we_ref, be_ref,
                wc_ref, bc_ref, eps_ref, wo_ref, bo_ref, out_ref):
    a = a_ref[0]
    h = jnp.dot(x_ref[0], wn_ref[...],
                preferred_element_type=jnp.float32) + bn_ref[...]
    we = we_ref[...]
    be = be_ref[...]
    ep = eps_ref[...]
    ms = [a * e_ref[0, k] for k in range(D_EDGE)]
    for l in range(L):
        ah = jnp.dot(a, h, preferred_element_type=jnp.float32)
        msg = be * ah
        for k in range(D_EDGE):
            mh = jnp.dot(ms[k], h, preferred_element_type=jnp.float32)
            msg = msg + we[k:k + 1, :] * mh
        pre = jnp.dot((1.0 + ep[l, 0]) * h + msg, wc_ref[l],
                      preferred_element_type=jnp.float32) + bc_ref[l:l + 1, :]
        h = jnp.maximum(pre, 0.0)
    hm = jnp.mean(h, axis=0, keepdims=True)
    val = jnp.dot(hm, wo_ref[...],
                  preferred_element_type=jnp.float32) + bo_ref[...]
    i = pl.program_id(0)
    out_ref[pl.ds(i, 1), :] = 1.0 + jnp.where(val >= 0.0, val, 0.01 * val)


def kernel(A, X, E, We, be, Wn, bn, Wc, bc, eps, Wo, bo):
    b, n, d_in = X.shape
    d_edge, d = We.shape
    n_layers = Wc.shape[0]
    d_in_pad = max(8, d_in)
    Xp = jnp.pad(X, ((0, 0), (0, 0), (0, d_in_pad - d_in)))
    Wnp = jnp.pad(Wn, ((0, d_in_pad - d_in), (0, 0)))
    Et = jnp.transpose(E, (0, 3, 1, 2))
    out = pl.pallas_call(
        _gnn_kernel,
        grid=(b,),
        in_specs=[
            pl.BlockSpec((1, n, n), lambda i: (i, 0, 0)),
            pl.BlockSpec((1, d_edge, n, n), lambda i: (i, 0, 0, 0)),
            pl.BlockSpec((1, n, d_in_pad), lambda i: (i, 0, 0)),
            pl.BlockSpec((d_in_pad, d), lambda i: (0, 0)),
            pl.BlockSpec((1, d), lambda i: (0, 0)),
            pl.BlockSpec((d_edge, d), lambda i: (0, 0)),
            pl.BlockSpec((1, d), lambda i: (0, 0)),
            pl.BlockSpec((n_layers, d, d), lambda i: (0, 0, 0)),
            pl.BlockSpec((n_layers, d), lambda i: (0, 0)),
            pl.BlockSpec((n_layers, 1), lambda i: (0, 0)),
            pl.BlockSpec((d, 1), lambda i: (0, 0)),
            pl.BlockSpec((1, 1), lambda i: (0, 0)),
        ],
        out_specs=pl.BlockSpec((b, 1), lambda i: (0, 0)),
        out_shape=jax.ShapeDtypeStruct((b, 1), jnp.float32),
    )(A, Et, Xp, Wnp, bn.reshape(1, d), We, be.reshape(1, d),
      Wc, bc, eps.reshape(n_layers, 1), Wo, bo.reshape(1, 1))
    return out


# speedup vs baseline: 2.9333x; 1.0660x over previous
import jax
import jax.numpy as jnp
from jax.experimental import pallas as pl


def _gnn_kernel(a_ref, e_ref, x_ref, wn_ref, bn_ref, we_ref, be_ref,
                wc_ref, bc_ref, eps_ref, wo_ref, bo_ref, out_ref):
    nb = a_ref.shape[0]
    d_edge = we_ref.shape[0]
    n_layers = wc_ref.shape[0]
    wn = wn_ref[...]
    bn = bn_ref[...]
    we = we_ref[...]
    be = be_ref[...]
    ep = eps_ref[...]
    wo = wo_ref[...]
    bo = bo_ref[...]
    for bidx in range(nb):
        a = a_ref[bidx]
        h = jnp.dot(x_ref[bidx], wn, preferred_element_type=jnp.float32) + bn
        ms = [a * e_ref[bidx, k] for k in range(d_edge)]
        for l in range(n_layers):
            ah = jnp.dot(a, h, preferred_element_type=jnp.float32)
            msg = be * ah
            for k in range(d_edge):
                mh = jnp.dot(ms[k], h, preferred_element_type=jnp.float32)
                msg = msg + we[k:k + 1, :] * mh
            pre = jnp.dot((1.0 + ep[l, 0]) * h + msg, wc_ref[l],
                          preferred_element_type=jnp.float32) + bc_ref[l:l + 1, :]
            h = jnp.maximum(pre, 0.0)
        hm = jnp.mean(h, axis=0, keepdims=True)
        val = jnp.dot(hm, wo, preferred_element_type=jnp.float32) + bo
        out_ref[bidx:bidx + 1, :] = 1.0 + jnp.where(val >= 0.0, val, 0.01 * val)


def kernel(A, X, E, We, be, Wn, bn, Wc, bc, eps, Wo, bo):
    b, n, d_in = X.shape
    d_edge, d = We.shape
    n_layers = Wc.shape[0]
    Et = jnp.transpose(E, (0, 3, 1, 2)).reshape(b, d_edge, n, n)
    out = pl.pallas_call(
        _gnn_kernel,
        out_shape=jax.ShapeDtypeStruct((b, 1), jnp.float32),
    )(A, Et, X, Wn, bn.reshape(1, d), We, be.reshape(1, d),
      Wc, bc, eps.reshape(n_layers, 1), Wo, bo.reshape(1, 1))
    return out
